# baseline (speedup 1.0000x reference)
import functools

import jax
import jax.numpy as jnp
from jax import lax
from jax.experimental import pallas as pl
from jax.experimental.pallas import tpu as pltpu
from jax.experimental.pallas import tpu_sc as plsc

IN_N = 2048
M_TOTAL = 16384
NC, NS = 2, 16
NW = NC * NS
ROWS_PER_W = M_TOTAL // NW
CH = 8
NCHUNK = ROWS_PER_W // CH
NBUF = 4


def _sc_copy(x_hbm, o_hbm, *scratch):
    wid = lax.axis_index("s") * NC + lax.axis_index("c")
    base = wid * ROWS_PER_W
    bufs = scratch[:NBUF]
    rsem, wsem = scratch[NBUF], scratch[NBUF + 1]

    def rd(c, b):
        return pltpu.make_async_copy(
            x_hbm.at[pl.ds(base + c * CH, CH), :], bufs[b], rsem.at[b])

    def wr(c, b):
        return pltpu.make_async_copy(
            bufs[b], o_hbm.at[pl.ds(base + c * CH, CH), :], wsem.at[b])

    for i in range(NBUF):
        rd(i, i).start()
    for c in range(NCHUNK):
        b = c % NBUF
        rd(c, b).wait()
        wr(c, b).start()
        j = c - (NBUF - 1)
        if 0 <= j < NCHUNK - NBUF:
            bj = j % NBUF
            wr(j, bj).wait()
            rd(j + NBUF, bj).start()
    for j in range(max(0, NCHUNK - NBUF), NCHUNK):
        wr(j, j % NBUF).wait()


def kernel(input):
    x = input.reshape(-1, IN_N)
    mesh = plsc.VectorSubcoreMesh(core_axis_name="c", subcore_axis_name="s")
    f = functools.partial(
        pl.kernel,
        mesh=mesh,
        out_type=jax.ShapeDtypeStruct((M_TOTAL, IN_N), jnp.float32),
        scratch_types=(
            [pltpu.VMEM((CH, IN_N), jnp.float32) for _ in range(NBUF)]
            + [pltpu.SemaphoreType.DMA((NBUF,)),
               pltpu.SemaphoreType.DMA((NBUF,))]
        ),
    )(_sc_copy)
    return f(x)


# speedup vs baseline: 1.2123x; 1.2123x over previous
import functools

import jax
import jax.numpy as jnp
from jax import lax
from jax.experimental import pallas as pl
from jax.experimental.pallas import tpu as pltpu
from jax.experimental.pallas import tpu_sc as plsc

IN_N = 2048
M_TOTAL = 16384
NC, NS = 2, 16
NW = NC * NS
ROWS_PER_W = M_TOTAL // NW
CH = 16
NCHUNK = ROWS_PER_W // CH
NBUF = 3


def _sc_copy(x_hbm, o_hbm, *scratch):
    wid = lax.axis_index("s") * NC + lax.axis_index("c")
    base = wid * ROWS_PER_W
    bufs = scratch[:NBUF]
    rsem, wsem = scratch[NBUF], scratch[NBUF + 1]

    def rd(c, b):
        return pltpu.make_async_copy(
            x_hbm.at[pl.ds(base + c * CH, CH), :], bufs[b], rsem.at[b])

    def wr(c, b):
        return pltpu.make_async_copy(
            bufs[b], o_hbm.at[pl.ds(base + c * CH, CH), :], wsem.at[b])

    for i in range(NBUF):
        rd(i, i).start()
    for c in range(NCHUNK):
        b = c % NBUF
        rd(c, b).wait()
        wr(c, b).start()
        j = c - (NBUF - 1)
        if 0 <= j < NCHUNK - NBUF:
            bj = j % NBUF
            wr(j, bj).wait()
            rd(j + NBUF, bj).start()
    for j in range(max(0, NCHUNK - NBUF), NCHUNK):
        wr(j, j % NBUF).wait()


def kernel(input):
    x = input.reshape(-1, IN_N)
    mesh = plsc.VectorSubcoreMesh(core_axis_name="c", subcore_axis_name="s")
    f = functools.partial(
        pl.kernel,
        mesh=mesh,
        out_type=jax.ShapeDtypeStruct((M_TOTAL, IN_N), jnp.float32),
        scratch_types=(
            [pltpu.VMEM((CH, IN_N), jnp.float32) for _ in range(NBUF)]
            + [pltpu.SemaphoreType.DMA((NBUF,)),
               pltpu.SemaphoreType.DMA((NBUF,))]
        ),
    )(_sc_copy)
    return f(x)
